# baseline (speedup 1.0000x reference)
import jax
import jax.numpy as jnp
from jax.experimental import pallas as pl

_G = 32
_D = 768
_HALF = _D // 2
_BATCH = 64
_BB = 8


def _body(row_ref, col_ref, out_ref):
    r = row_ref[...]
    c = col_ref[...]
    re = jnp.broadcast_to(r[:, None, :], (_G, _G, _HALF))
    ce = jnp.broadcast_to(c[None, :, :], (_G, _G, _HALF))
    pos = jnp.concatenate([re, ce], axis=-1)
    out_ref[...] = jnp.broadcast_to(pos[None], (_BB, _G, _G, _D))


def kernel(batch_size, row_table, col_table):
    del batch_size
    grid = (_BATCH // _BB,)
    return pl.pallas_call(
        _body,
        grid=grid,
        in_specs=[
            pl.BlockSpec((_G, _HALF), lambda b: (0, 0)),
            pl.BlockSpec((_G, _HALF), lambda b: (0, 0)),
        ],
        out_specs=pl.BlockSpec((_BB, _G, _G, _D), lambda b: (b, 0, 0, 0)),
        out_shape=jax.ShapeDtypeStruct((_BATCH, _G, _G, _D), jnp.float32),
    )(row_table, col_table)


# speedup vs baseline: 1.0328x; 1.0328x over previous
import jax
import jax.numpy as jnp
from jax.experimental import pallas as pl

_G = 32
_D = 768
_HALF = _D // 2
_BATCH = 64
_BB = 2


def _body(row_ref, col_ref, out_ref):
    r = row_ref[...]
    c = col_ref[...]
    re = jnp.broadcast_to(r[:, None, :], (_G, _G, _HALF))
    ce = jnp.broadcast_to(c[None, :, :], (_G, _G, _HALF))
    pos = jnp.concatenate([re, ce], axis=-1)
    out_ref[...] = jnp.broadcast_to(pos[None], (_BB, _G, _G, _D))


def kernel(batch_size, row_table, col_table):
    del batch_size
    grid = (_BATCH // _BB,)
    return pl.pallas_call(
        _body,
        grid=grid,
        in_specs=[
            pl.BlockSpec((_G, _HALF), lambda b: (0, 0)),
            pl.BlockSpec((_G, _HALF), lambda b: (0, 0)),
        ],
        out_specs=pl.BlockSpec((_BB, _G, _G, _D), lambda b: (b, 0, 0, 0)),
        out_shape=jax.ShapeDtypeStruct((_BATCH, _G, _G, _D), jnp.float32),
    )(row_table, col_table)
